# baseline (speedup 1.0000x reference)
import functools

import jax
import jax.numpy as jnp
from jax import lax
from jax.experimental import pallas as pl
from jax.experimental.pallas import tpu as pltpu
from jax.experimental.pallas import tpu_sc as plsc

D = 768
LANES = 16
NV = D // LANES
R = 32
NBUF = 2


def _build_sc_kernel(n_rows):
    info = plsc.get_sparse_core_info()
    nc, ns = info.num_cores, info.num_subcores
    nw = nc * ns
    rows_w = n_rows // nw
    nchunks = rows_w // R
    mesh = plsc.VectorSubcoreMesh(core_axis_name="c", subcore_axis_name="s")

    @functools.partial(
        pl.kernel,
        mesh=mesh,
        out_type=jax.ShapeDtypeStruct((n_rows, D), jnp.float32),
        scratch_types=[
            pltpu.VMEM((NBUF, R, D), jnp.float32),
            pltpu.VMEM((NBUF, R, D), jnp.float32),
            pltpu.VMEM((NBUF, R), jnp.int32),
            pltpu.SemaphoreType.DMA,
            pltpu.SemaphoreType.DMA,
            pltpu.SemaphoreType.DMA,
            pltpu.SemaphoreType.DMA,
            pltpu.SemaphoreType.DMA,
            pltpu.SemaphoreType.DMA,
        ],
    )
    def sc_kernel(emb_hbm, idx_hbm, tbl_hbm, out_hbm, emb_v, tbl_v, idx_v,
                  sem_e0, sem_e1, sem_t0, sem_t1, sem_o0, sem_o1):
        sem_e = (sem_e0, sem_e1)
        sem_t = (sem_t0, sem_t1)
        sem_o = (sem_o0, sem_o1)
        wid = lax.axis_index("s") * nc + lax.axis_index("c")
        base = wid * rows_w

        def start_in(c, b):
            rbase = base + c * R
            pltpu.sync_copy(idx_hbm.at[pl.ds(rbase, R)], idx_v.at[b])
            pltpu.async_copy(emb_hbm.at[pl.ds(rbase, R)], emb_v.at[b],
                             sem_e[b])
            pltpu.async_copy(tbl_hbm.at[idx_v.at[b]], tbl_v.at[b], sem_t[b])

        def wait_in(c, b):
            rbase = base + c * R
            pltpu.make_async_copy(emb_hbm.at[pl.ds(rbase, R)], emb_v.at[b],
                                  sem_e[b]).wait()
            pltpu.make_async_copy(tbl_hbm.at[idx_v.at[b]], tbl_v.at[b],
                                  sem_t[b]).wait()

        def start_out(c, b):
            rbase = base + c * R
            pltpu.async_copy(emb_v.at[b], out_hbm.at[pl.ds(rbase, R)],
                             sem_o[b])

        def wait_out(c, b):
            rbase = base + c * R
            pltpu.make_async_copy(emb_v.at[b], out_hbm.at[pl.ds(rbase, R)],
                                  sem_o[b]).wait()

        start_in(0, 0)

        def chunk_pair(c2, carry):
            for b in range(NBUF):
                c = c2 * NBUF + b
                nb = 1 - b

                @pl.when(c + 1 < nchunks)
                def _prefetch():
                    @pl.when(c >= 1)
                    def _drain():
                        wait_out(c - 1, nb)
                    start_in(c + 1, nb)

                wait_in(c, b)

                def row_body(r, rc):
                    for v in range(NV):
                        sl = pl.ds(v * LANES, LANES)
                        emb_v[b, r, sl] = emb_v[b, r, sl] + tbl_v[b, r, sl] * 0.1
                    return rc

                lax.fori_loop(0, R, row_body, 0)
                start_out(c, b)
            return carry

        lax.fori_loop(0, nchunks // NBUF, chunk_pair, 0)
        wait_out(nchunks - 2, 0)
        wait_out(nchunks - 1, 1)

    return sc_kernel


def kernel(embeddings, section_indices, section_table):
    b, t, d = embeddings.shape
    n = b * t
    emb2d = embeddings.reshape(n, d)
    idx = section_indices.reshape(n).astype(jnp.int32)
    out = _build_sc_kernel(n)(emb2d, idx, section_table)
    return out.reshape(b, t, d)
